# Initial kernel scaffold; baseline (speedup 1.0000x reference)
import jax
import jax.numpy as jnp
from jax.experimental import pallas as pl


def kernel(nodes, edges, i, j, params):
    raise NotImplementedError("write your pallas kernel here")


# speedup vs baseline: 2.0309x; 2.0309x over previous
import functools

import jax
import jax.numpy as jnp
from jax import lax
from jax.experimental import pallas as pl
from jax.experimental.pallas import tpu as pltpu
from jax.experimental.pallas import tpu_sc as plsc

_EPS = 1e-5
_BF = jnp.bfloat16


def _sc_gather(table, idx):
    V, W = table.shape
    B = idx.shape[0]
    NW = 32
    b_per_w = B // NW
    CH = 80
    n_ch = b_per_w // CH
    mesh = plsc.VectorSubcoreMesh(core_axis_name="c", subcore_axis_name="s")

    @functools.partial(
        pl.kernel, mesh=mesh,
        out_type=jax.ShapeDtypeStruct((B, W), table.dtype),
        scratch_types=[
            pltpu.VMEM((b_per_w,), jnp.int32),
            pltpu.VMEM((CH, W), table.dtype),
            pltpu.SemaphoreType.DMA,
        ],
    )
    def k(table_hbm, idx_hbm, out_hbm, idx_v, rows_v, sem):
        wid = lax.axis_index("s") * 2 + lax.axis_index("c")
        base = wid * b_per_w
        pltpu.sync_copy(idx_hbm.at[pl.ds(base, b_per_w)], idx_v)

        def body(c, carry):
            off = c * CH
            pltpu.async_copy(
                table_hbm.at[idx_v.at[pl.ds(off, CH)]], rows_v, sem).wait()
            pltpu.sync_copy(rows_v, out_hbm.at[pl.ds(base + off, CH)])
            return carry

        lax.fori_loop(0, n_ch, body, 0)

    return k(table, idx)


def _ln1(x, s, o):
    mu = jnp.mean(x, axis=-1, keepdims=True)
    var = jnp.mean(x * x, axis=-1, keepdims=True) - mu * mu
    return (x - mu) * lax.rsqrt(var + _EPS) * s + o


def _dot(x, w):
    return jnp.dot(x.astype(_BF), w, preferred_element_type=jnp.float32)


def _unpack(u):
    lo = lax.bitcast_convert_type(u << 16, jnp.float32)
    hi = lax.bitcast_convert_type(u & jnp.int32(-65536), jnp.float32)
    return lo, hi


def _tc_body(ni_ref, nj_ref, e_ref, nd_ref,
             s0ae, s0ao, s0be, s0bo, s0e, o0ae, o0ao, o0be, o0bo, o0e,
             w0ae, w0ao, w0be, w0bo, w0e, b0,
             s1m, o1m, w1, b1,
             slm, olm, wl, bl,
             s1n, o1n, wtr1, btr1,
             s2a, s2b, o2a, o2b, wupa, wupb, bup,
             s3ae, s3ao, s3be, s3bo, o3ae, o3ao, o3be, o3bo,
             wt2ae, wt2ao, wt2be, wt2bo, bt2,
             s4e, s4t, o4e, o4t, wue, wut, bue,
             nup_ref, eup_ref):
    nie, nio = _unpack(ni_ref[...])
    nje, njo = _unpack(nj_ref[...])
    e = e_ref[...]
    C = 2 * nie.shape[1]
    DE = e.shape[1]
    BE = nie.shape[0]
    BN = nup_ref.shape[0]

    sum_ni = jnp.sum(nie, -1, keepdims=True) + jnp.sum(nio, -1, keepdims=True)
    sum_nj = jnp.sum(nje, -1, keepdims=True) + jnp.sum(njo, -1, keepdims=True)
    ssq_ni = (jnp.sum(nie * nie, -1, keepdims=True)
              + jnp.sum(nio * nio, -1, keepdims=True))
    ssq_nj = (jnp.sum(nje * nje, -1, keepdims=True)
              + jnp.sum(njo * njo, -1, keepdims=True))
    sum_e = jnp.sum(e, -1, keepdims=True)
    ssq_e = jnp.sum(e * e, -1, keepdims=True)

    d0 = 2 * C + DE
    mu = (sum_ni + sum_nj + sum_e) / d0
    inv = lax.rsqrt((ssq_ni + ssq_nj + ssq_e) / d0 - mu * mu + _EPS)
    x = (_dot((nie - mu) * inv * s0ae[...] + o0ae[...], w0ae[...])
         + _dot((nio - mu) * inv * s0ao[...] + o0ao[...], w0ao[...])
         + _dot((nje - mu) * inv * s0be[...] + o0be[...], w0be[...])
         + _dot((njo - mu) * inv * s0bo[...] + o0bo[...], w0bo[...])
         + _dot((e - mu) * inv * s0e[...] + o0e[...], w0e[...]) + b0[...])
    x = jax.nn.relu(x)
    x = jax.nn.relu(_dot(_ln1(x, s1m[...], o1m[...]), w1[...]) + b1[...])
    x = _dot(_ln1(x, slm[...], olm[...]), wl[...]) + bl[...]
    m = jax.nn.relu(x)
    m_i = jnp.sum(m.reshape(BN, BE // BN, C), axis=1)

    nd = nd_ref[...]
    h = jax.nn.relu(_dot(_ln1(nd, s1n[...], o1n[...]), wtr1[...]) + btr1[...])
    d2 = 2 * C
    mu2 = (jnp.sum(h, -1, keepdims=True) + jnp.sum(m_i, -1, keepdims=True)) / d2
    ex22 = (jnp.sum(h * h, -1, keepdims=True)
            + jnp.sum(m_i * m_i, -1, keepdims=True)) / d2
    inv2 = lax.rsqrt(ex22 - mu2 * mu2 + _EPS)
    nup = jax.nn.relu(
        _dot((h - mu2) * inv2 * s2a[...] + o2a[...], wupa[...])
        + _dot((m_i - mu2) * inv2 * s2b[...] + o2b[...], wupb[...]) + bup[...])
    nup_ref[...] = nup

    mu3 = (sum_ni + sum_nj) / d2
    inv3 = lax.rsqrt((ssq_ni + ssq_nj) / d2 - mu3 * mu3 + _EPS)
    t = jax.nn.relu(
        _dot((nie - mu3) * inv3 * s3ae[...] + o3ae[...], wt2ae[...])
        + _dot((nio - mu3) * inv3 * s3ao[...] + o3ao[...], wt2ao[...])
        + _dot((nje - mu3) * inv3 * s3be[...] + o3be[...], wt2be[...])
        + _dot((njo - mu3) * inv3 * s3bo[...] + o3bo[...], wt2bo[...])
        + bt2[...])
    d4 = DE + C
    mu4 = (sum_e + jnp.sum(t, -1, keepdims=True)) / d4
    ex24 = (ssq_e + jnp.sum(t * t, -1, keepdims=True)) / d4
    inv4 = lax.rsqrt(ex24 - mu4 * mu4 + _EPS)
    eup = jax.nn.relu(
        _dot((e - mu4) * inv4 * s4e[...] + o4e[...], wue[...])
        + _dot((t - mu4) * inv4 * s4t[...] + o4t[...], wut[...]) + bue[...])
    eup_ref[...] = eup


def kernel(nodes, edges, i, j, params):
    N, C = nodes.shape
    E, DE = edges.shape
    P = params

    idx = jnp.concatenate([i, j]).astype(jnp.int32)
    packed = lax.bitcast_convert_type(
        nodes.astype(_BF).reshape(N, C // 2, 2), jnp.int32)
    gath = _sc_gather(packed, idx)

    r = lambda v: v.reshape(1, -1)
    bf = lambda w: w.astype(_BF)
    s0, o0 = P["msg_ln0"]["s"], P["msg_ln0"]["o"]
    w0, b0 = P["msg_l0"]["w"], P["msg_l0"]["b"]
    s2, o2 = P["ln2"]["s"], P["ln2"]["o"]
    wup, bup = P["up"]["w"], P["up"]["b"]
    s3, o3 = P["ln3"]["s"], P["ln3"]["o"]
    wt2, bt2 = P["tr2"]["w"], P["tr2"]["b"]
    s4, o4 = P["ln4"]["s"], P["ln4"]["o"]
    wue, bue = P["eup"]["w"], P["eup"]["b"]
    s0a, s0b, s0e = s0[:C], s0[C:2 * C], s0[2 * C:]
    o0a, o0b, o0e = o0[:C], o0[C:2 * C], o0[2 * C:]
    w0a, w0b, w0e = w0[:C], w0[C:2 * C], w0[2 * C:]
    s3a, s3b, o3a, o3b = s3[:C], s3[C:], o3[:C], o3[C:]
    wt2a, wt2b = wt2[:C], wt2[C:]
    plist = [
        r(s0a[0::2]), r(s0a[1::2]), r(s0b[0::2]), r(s0b[1::2]), r(s0e),
        r(o0a[0::2]), r(o0a[1::2]), r(o0b[0::2]), r(o0b[1::2]), r(o0e),
        bf(w0a[0::2]), bf(w0a[1::2]), bf(w0b[0::2]), bf(w0b[1::2]),
        bf(w0e), r(b0),
        r(P["msg_ln1"]["s"]), r(P["msg_ln1"]["o"]),
        bf(P["msg_l1"]["w"]), r(P["msg_l1"]["b"]),
        r(P["msg_lnl"]["s"]), r(P["msg_lnl"]["o"]),
        bf(P["msg_ll"]["w"]), r(P["msg_ll"]["b"]),
        r(P["ln1"]["s"]), r(P["ln1"]["o"]),
        bf(P["tr1"]["w"]), r(P["tr1"]["b"]),
        r(s2[:C]), r(s2[C:]), r(o2[:C]), r(o2[C:]),
        bf(wup[:C]), bf(wup[C:]), r(bup),
        r(s3a[0::2]), r(s3a[1::2]), r(s3b[0::2]), r(s3b[1::2]),
        r(o3a[0::2]), r(o3a[1::2]), r(o3b[0::2]), r(o3b[1::2]),
        bf(wt2a[0::2]), bf(wt2a[1::2]), bf(wt2b[0::2]), bf(wt2b[1::2]),
        r(bt2),
        r(s4[:DE]), r(s4[DE:]), r(o4[:DE]), r(o4[DE:]),
        bf(wue[:DE]), bf(wue[DE:]), r(bue),
    ]

    BE = 3200
    BN = BE // (E // N)
    G = E // BE
    W = C // 2
    full = lambda p: pl.BlockSpec(p.shape, lambda b: tuple(0 for _ in p.shape))
    in_specs = [
        pl.BlockSpec((BE, W), lambda b: (b, 0)),
        pl.BlockSpec((BE, W), lambda b: (b + G, 0)),
        pl.BlockSpec((BE, DE), lambda b: (b, 0)),
        pl.BlockSpec((BN, C), lambda b: (b, 0)),
    ] + [full(p) for p in plist]
    out_specs = [
        pl.BlockSpec((BN, C), lambda b: (b, 0)),
        pl.BlockSpec((BE, C), lambda b: (b, 0)),
    ]
    n_up, e_up = pl.pallas_call(
        _tc_body,
        grid=(G,),
        in_specs=in_specs,
        out_specs=out_specs,
        out_shape=[
            jax.ShapeDtypeStruct((N, C), jnp.float32),
            jax.ShapeDtypeStruct((E, C), jnp.float32),
        ],
    )(gath, gath, edges, nodes, *plist)
    return (n_up, e_up)


# speedup vs baseline: 2.8226x; 1.3898x over previous
import functools

import jax
import jax.numpy as jnp
from jax import lax
from jax.experimental import pallas as pl
from jax.experimental.pallas import tpu as pltpu
from jax.experimental.pallas import tpu_sc as plsc

_EPS = 1e-5
_BF = jnp.bfloat16


def _sc_gather(table, idx):
    V, W = table.shape
    B = idx.shape[0]
    NW = 32
    b_per_w = B // NW
    CH = 80
    n_ch = b_per_w // CH
    mesh = plsc.VectorSubcoreMesh(core_axis_name="c", subcore_axis_name="s")

    @functools.partial(
        pl.kernel, mesh=mesh,
        out_type=jax.ShapeDtypeStruct((B, W), table.dtype),
        scratch_types=[
            pltpu.VMEM((b_per_w,), jnp.int32),
            pltpu.VMEM((CH, W), table.dtype),
            pltpu.SemaphoreType.DMA,
        ],
    )
    def k(table_hbm, idx_hbm, out_hbm, idx_v, rows_v, sem):
        wid = lax.axis_index("s") * 2 + lax.axis_index("c")
        base = wid * b_per_w
        pltpu.sync_copy(idx_hbm.at[pl.ds(base, b_per_w)], idx_v)

        def body(c, carry):
            off = c * CH
            pltpu.async_copy(
                table_hbm.at[idx_v.at[pl.ds(off, CH)]], rows_v, sem).wait()
            pltpu.sync_copy(rows_v, out_hbm.at[pl.ds(base + off, CH)])
            return carry

        lax.fori_loop(0, n_ch, body, 0)

    return k(table, idx)


def _ln1(x, s, o):
    mu = jnp.mean(x, axis=-1, keepdims=True)
    var = jnp.mean(x * x, axis=-1, keepdims=True) - mu * mu
    return (x - mu) * lax.rsqrt(var + _EPS) * s + o


def _dot(x, w):
    return jnp.dot(x.astype(_BF), w, preferred_element_type=jnp.float32)


def _unpack(u):
    lo = lax.bitcast_convert_type(u << 16, jnp.float32)
    hi = lax.bitcast_convert_type(u & jnp.int32(-65536), jnp.float32)
    return jnp.concatenate([lo, hi], axis=-1)


def _tc_body(ni_ref, nj_ref, e_ref, nd_ref,
             s0a, s0b, s0e, o0a, o0b, o0e, w0a, w0b, w0e, b0,
             s1m, o1m, w1, b1,
             slm, olm, wl, bl,
             s1n, o1n, wtr1, btr1,
             s2a, s2b, o2a, o2b, wupa, wupb, bup,
             s3a, s3b, o3a, o3b, wt2a, wt2b, bt2,
             s4e, s4t, o4e, o4t, wue, wut, bue,
             nup_ref, eup_ref):
    ni = _unpack(ni_ref[...])
    nj = _unpack(nj_ref[...])
    e = e_ref[...]
    C = ni.shape[1]
    DE = e.shape[1]
    BE = ni.shape[0]
    BN = nup_ref.shape[0]

    sum_ni = jnp.sum(ni, -1, keepdims=True)
    sum_nj = jnp.sum(nj, -1, keepdims=True)
    ssq_ni = jnp.sum(ni * ni, -1, keepdims=True)
    ssq_nj = jnp.sum(nj * nj, -1, keepdims=True)
    sum_e = jnp.sum(e, -1, keepdims=True)
    ssq_e = jnp.sum(e * e, -1, keepdims=True)

    d0 = 2 * C + DE
    mu = (sum_ni + sum_nj + sum_e) / d0
    inv = lax.rsqrt((ssq_ni + ssq_nj + ssq_e) / d0 - mu * mu + _EPS)
    x = (_dot((ni - mu) * inv * s0a[...] + o0a[...], w0a[...])
         + _dot((nj - mu) * inv * s0b[...] + o0b[...], w0b[...])
         + _dot((e - mu) * inv * s0e[...] + o0e[...], w0e[...]) + b0[...])
    x = jax.nn.relu(x)
    x = jax.nn.relu(_dot(_ln1(x, s1m[...], o1m[...]), w1[...]) + b1[...])
    x = _dot(_ln1(x, slm[...], olm[...]), wl[...]) + bl[...]
    m = jax.nn.relu(x)
    m_i = jnp.sum(m.reshape(BN, BE // BN, C), axis=1)

    nd = nd_ref[...]
    h = jax.nn.relu(_dot(_ln1(nd, s1n[...], o1n[...]), wtr1[...]) + btr1[...])
    d2 = 2 * C
    mu2 = (jnp.sum(h, -1, keepdims=True) + jnp.sum(m_i, -1, keepdims=True)) / d2
    ex22 = (jnp.sum(h * h, -1, keepdims=True)
            + jnp.sum(m_i * m_i, -1, keepdims=True)) / d2
    inv2 = lax.rsqrt(ex22 - mu2 * mu2 + _EPS)
    nup = jax.nn.relu(
        _dot((h - mu2) * inv2 * s2a[...] + o2a[...], wupa[...])
        + _dot((m_i - mu2) * inv2 * s2b[...] + o2b[...], wupb[...]) + bup[...])
    nup_ref[...] = nup

    mu3 = (sum_ni + sum_nj) / d2
    inv3 = lax.rsqrt((ssq_ni + ssq_nj) / d2 - mu3 * mu3 + _EPS)
    t = jax.nn.relu(
        _dot((ni - mu3) * inv3 * s3a[...] + o3a[...], wt2a[...])
        + _dot((nj - mu3) * inv3 * s3b[...] + o3b[...], wt2b[...]) + bt2[...])
    d4 = DE + C
    mu4 = (sum_e + jnp.sum(t, -1, keepdims=True)) / d4
    ex24 = (ssq_e + jnp.sum(t * t, -1, keepdims=True)) / d4
    inv4 = lax.rsqrt(ex24 - mu4 * mu4 + _EPS)
    eup = jax.nn.relu(
        _dot((e - mu4) * inv4 * s4e[...] + o4e[...], wue[...])
        + _dot((t - mu4) * inv4 * s4t[...] + o4t[...], wut[...]) + bue[...])
    eup_ref[...] = eup


def kernel(nodes, edges, i, j, params):
    N, C = nodes.shape
    E, DE = edges.shape
    H = C // 2
    P = params

    idx = jnp.concatenate([i, j]).astype(jnp.int32)
    nb = nodes.astype(_BF)
    packed = lax.bitcast_convert_type(
        jnp.stack([nb[:, :H], nb[:, H:]], axis=-1), jnp.int32)
    gath = _sc_gather(packed, idx)

    r = lambda v: v.reshape(1, -1)
    bf = lambda w: w.astype(_BF)
    s0, o0 = P["msg_ln0"]["s"], P["msg_ln0"]["o"]
    w0, b0 = P["msg_l0"]["w"], P["msg_l0"]["b"]
    s2, o2 = P["ln2"]["s"], P["ln2"]["o"]
    wup, bup = P["up"]["w"], P["up"]["b"]
    s3, o3 = P["ln3"]["s"], P["ln3"]["o"]
    wt2, bt2 = P["tr2"]["w"], P["tr2"]["b"]
    s4, o4 = P["ln4"]["s"], P["ln4"]["o"]
    wue, bue = P["eup"]["w"], P["eup"]["b"]
    plist = [
        r(s0[:C]), r(s0[C:2 * C]), r(s0[2 * C:]),
        r(o0[:C]), r(o0[C:2 * C]), r(o0[2 * C:]),
        bf(w0[:C]), bf(w0[C:2 * C]), bf(w0[2 * C:]), r(b0),
        r(P["msg_ln1"]["s"]), r(P["msg_ln1"]["o"]),
        bf(P["msg_l1"]["w"]), r(P["msg_l1"]["b"]),
        r(P["msg_lnl"]["s"]), r(P["msg_lnl"]["o"]),
        bf(P["msg_ll"]["w"]), r(P["msg_ll"]["b"]),
        r(P["ln1"]["s"]), r(P["ln1"]["o"]),
        bf(P["tr1"]["w"]), r(P["tr1"]["b"]),
        r(s2[:C]), r(s2[C:]), r(o2[:C]), r(o2[C:]),
        bf(wup[:C]), bf(wup[C:]), r(bup),
        r(s3[:C]), r(s3[C:]), r(o3[:C]), r(o3[C:]),
        bf(wt2[:C]), bf(wt2[C:]), r(bt2),
        r(s4[:DE]), r(s4[DE:]), r(o4[:DE]), r(o4[DE:]),
        bf(wue[:DE]), bf(wue[DE:]), r(bue),
    ]

    BE = 3200
    BN = BE // (E // N)
    G = E // BE
    full = lambda p: pl.BlockSpec(p.shape, lambda b: tuple(0 for _ in p.shape))
    in_specs = [
        pl.BlockSpec((BE, H), lambda b: (b, 0)),
        pl.BlockSpec((BE, H), lambda b: (b + G, 0)),
        pl.BlockSpec((BE, DE), lambda b: (b, 0)),
        pl.BlockSpec((BN, C), lambda b: (b, 0)),
    ] + [full(p) for p in plist]
    out_specs = [
        pl.BlockSpec((BN, C), lambda b: (b, 0)),
        pl.BlockSpec((BE, C), lambda b: (b, 0)),
    ]
    n_up, e_up = pl.pallas_call(
        _tc_body,
        grid=(G,),
        in_specs=in_specs,
        out_specs=out_specs,
        out_shape=[
            jax.ShapeDtypeStruct((N, C), jnp.float32),
            jax.ShapeDtypeStruct((E, C), jnp.float32),
        ],
    )(gath, gath, edges, nodes, *plist)
    return (n_up, e_up)


# speedup vs baseline: 3.1991x; 1.1334x over previous
import functools

import jax
import jax.numpy as jnp
from jax import lax
from jax.experimental import pallas as pl
from jax.experimental.pallas import tpu as pltpu
from jax.experimental.pallas import tpu_sc as plsc

_EPS = 1e-5
_BF = jnp.bfloat16


def _sc_gather(table, idx):
    V, W = table.shape
    B = idx.shape[0]
    NW = 32
    b_per_w = B // NW
    CH = 80
    n_ch = b_per_w // CH
    mesh = plsc.VectorSubcoreMesh(core_axis_name="c", subcore_axis_name="s")

    @functools.partial(
        pl.kernel, mesh=mesh,
        out_type=jax.ShapeDtypeStruct((B, W), table.dtype),
        scratch_types=[
            pltpu.VMEM((b_per_w,), jnp.int32),
            pltpu.VMEM((CH, W), table.dtype),
            pltpu.SemaphoreType.DMA,
        ],
    )
    def k(table_hbm, idx_hbm, out_hbm, idx_v, rows_v, sem):
        wid = lax.axis_index("s") * 2 + lax.axis_index("c")
        base = wid * b_per_w
        pltpu.sync_copy(idx_hbm.at[pl.ds(base, b_per_w)], idx_v)

        def body(c, carry):
            off = c * CH
            pltpu.async_copy(
                table_hbm.at[idx_v.at[pl.ds(off, CH)]], rows_v, sem).wait()
            pltpu.sync_copy(rows_v, out_hbm.at[pl.ds(base + off, CH)])
            return carry

        lax.fori_loop(0, n_ch, body, 0)

    return k(table, idx)


def _dot(x, w):
    return jnp.dot(x.astype(_BF), w, preferred_element_type=jnp.float32)


def _unpack(u):
    lo = lax.bitcast_convert_type(u << 16, jnp.float32)
    hi = lax.bitcast_convert_type(u & jnp.int32(-65536), jnp.float32)
    return jnp.concatenate([lo, hi], axis=-1)


def _moments(sums, ssqs, d):
    mu = sums / d
    inv = lax.rsqrt(ssqs / d - mu * mu + _EPS)
    return mu * inv, inv


def _sum2(x):
    return jnp.sum(x, -1, keepdims=True), jnp.sum(x * x, -1, keepdims=True)


def _tc_body(ni_ref, nj_ref, e_ref, nd_ref,
             w0a, w0b, w0e, u0, v0,
             w1, u1, v1,
             wl, ul, vl,
             wtr1, ut1, vt1,
             wupa, wupb, u2, v2,
             wt2a, wt2b, u3, v3,
             wue, wut, u4, v4,
             nup_ref, eup_ref):
    ni = _unpack(ni_ref[...])
    nj = _unpack(nj_ref[...])
    e = e_ref[...]
    C = ni.shape[1]
    DE = e.shape[1]
    BE = ni.shape[0]
    BN = nup_ref.shape[0]

    sum_ni, ssq_ni = _sum2(ni)
    sum_nj, ssq_nj = _sum2(nj)
    sum_e, ssq_e = _sum2(e)

    mi0, inv0 = _moments(sum_ni + sum_nj + sum_e, ssq_ni + ssq_nj + ssq_e,
                         2 * C + DE)
    x = (_dot(ni, w0a[...]) + _dot(nj, w0b[...]) + _dot(e, w0e[...]))
    x = jax.nn.relu(x * inv0 - mi0 * u0[...] + v0[...])
    s, q = _sum2(x)
    mi1, inv1 = _moments(s, q, C)
    x = jax.nn.relu(_dot(x, w1[...]) * inv1 - mi1 * u1[...] + v1[...])
    s, q = _sum2(x)
    mil, invl = _moments(s, q, C)
    m = jax.nn.relu(_dot(x, wl[...]) * invl - mil * ul[...] + vl[...])
    m_i = jnp.sum(m.reshape(BN, BE // BN, C), axis=1)

    nd = nd_ref[...]
    s, q = _sum2(nd)
    min_, invn = _moments(s, q, C)
    h = jax.nn.relu(_dot(nd, wtr1[...]) * invn - min_ * ut1[...] + vt1[...])
    sh, qh = _sum2(h)
    sm, qm = _sum2(m_i)
    mi2, inv2 = _moments(sh + sm, qh + qm, 2 * C)
    nup = jax.nn.relu(
        (_dot(h, wupa[...]) + _dot(m_i, wupb[...])) * inv2
        - mi2 * u2[...] + v2[...])
    nup_ref[...] = nup

    mi3, inv3 = _moments(sum_ni + sum_nj, ssq_ni + ssq_nj, 2 * C)
    t = jax.nn.relu(
        (_dot(ni, wt2a[...]) + _dot(nj, wt2b[...])) * inv3
        - mi3 * u3[...] + v3[...])
    st, qt = _sum2(t)
    mi4, inv4 = _moments(sum_e + st, ssq_e + qt, DE + C)
    eup = jax.nn.relu(
        (_dot(e, wue[...]) + _dot(t, wut[...])) * inv4
        - mi4 * u4[...] + v4[...])
    eup_ref[...] = eup


def _fold(s, o, w, b):
    wp = w * s[:, None]
    u = jnp.sum(wp, axis=0).reshape(1, -1)
    v = (o @ w + b).reshape(1, -1)
    return wp, u, v


def kernel(nodes, edges, i, j, params):
    N, C = nodes.shape
    E, DE = edges.shape
    H = C // 2
    P = params

    idx = jnp.concatenate([i, j]).astype(jnp.int32)
    nb = nodes.astype(_BF)
    packed = lax.bitcast_convert_type(
        jnp.stack([nb[:, :H], nb[:, H:]], axis=-1), jnp.int32)
    gath = _sc_gather(packed, idx)

    bf = lambda w: w.astype(_BF)
    w0p, u0, v0 = _fold(P["msg_ln0"]["s"], P["msg_ln0"]["o"],
                        P["msg_l0"]["w"], P["msg_l0"]["b"])
    w1p, u1, v1 = _fold(P["msg_ln1"]["s"], P["msg_ln1"]["o"],
                        P["msg_l1"]["w"], P["msg_l1"]["b"])
    wlp, ul, vl = _fold(P["msg_lnl"]["s"], P["msg_lnl"]["o"],
                        P["msg_ll"]["w"], P["msg_ll"]["b"])
    wt1p, ut1, vt1 = _fold(P["ln1"]["s"], P["ln1"]["o"],
                           P["tr1"]["w"], P["tr1"]["b"])
    wupp, u2, v2 = _fold(P["ln2"]["s"], P["ln2"]["o"],
                         P["up"]["w"], P["up"]["b"])
    wt2p, u3, v3 = _fold(P["ln3"]["s"], P["ln3"]["o"],
                         P["tr2"]["w"], P["tr2"]["b"])
    wuep, u4, v4 = _fold(P["ln4"]["s"], P["ln4"]["o"],
                         P["eup"]["w"], P["eup"]["b"])
    plist = [
        bf(w0p[:C]), bf(w0p[C:2 * C]), bf(w0p[2 * C:]), u0, v0,
        bf(w1p), u1, v1,
        bf(wlp), ul, vl,
        bf(wt1p), ut1, vt1,
        bf(wupp[:C]), bf(wupp[C:]), u2, v2,
        bf(wt2p[:C]), bf(wt2p[C:]), u3, v3,
        bf(wuep[:DE]), bf(wuep[DE:]), u4, v4,
    ]

    BE = 3200
    BN = BE // (E // N)
    G = E // BE
    full = lambda p: pl.BlockSpec(p.shape, lambda b: tuple(0 for _ in p.shape))
    in_specs = [
        pl.BlockSpec((BE, H), lambda b: (b, 0)),
        pl.BlockSpec((BE, H), lambda b: (b + G, 0)),
        pl.BlockSpec((BE, DE), lambda b: (b, 0)),
        pl.BlockSpec((BN, C), lambda b: (b, 0)),
    ] + [full(p) for p in plist]
    out_specs = [
        pl.BlockSpec((BN, C), lambda b: (b, 0)),
        pl.BlockSpec((BE, C), lambda b: (b, 0)),
    ]
    n_up, e_up = pl.pallas_call(
        _tc_body,
        grid=(G,),
        in_specs=in_specs,
        out_specs=out_specs,
        out_shape=[
            jax.ShapeDtypeStruct((N, C), jnp.float32),
            jax.ShapeDtypeStruct((E, C), jnp.float32),
        ],
    )(gath, gath, edges, nodes, *plist)
    return (n_up, e_up)


# speedup vs baseline: 3.5546x; 1.1111x over previous
import functools

import jax
import jax.numpy as jnp
from jax import lax
from jax.experimental import pallas as pl
from jax.experimental.pallas import tpu as pltpu
from jax.experimental.pallas import tpu_sc as plsc

_EPS = 1e-5
_BF = jnp.bfloat16


def _sc_gather(table, idx):
    V, W = table.shape
    B = idx.shape[0]
    NW = 32
    b_per_w = B // NW
    CH = 80
    n_ch = b_per_w // CH
    mesh = plsc.VectorSubcoreMesh(core_axis_name="c", subcore_axis_name="s")

    @functools.partial(
        pl.kernel, mesh=mesh,
        out_type=jax.ShapeDtypeStruct((B, W), table.dtype),
        scratch_types=[
            pltpu.VMEM((b_per_w,), jnp.int32),
            pltpu.VMEM((CH, W), table.dtype),
            pltpu.SemaphoreType.DMA,
        ],
    )
    def k(table_hbm, idx_hbm, out_hbm, idx_v, rows_v, sem):
        wid = lax.axis_index("s") * 2 + lax.axis_index("c")
        base = wid * b_per_w
        pltpu.sync_copy(idx_hbm.at[pl.ds(base, b_per_w)], idx_v)

        def body(c, carry):
            off = c * CH
            pltpu.async_copy(
                table_hbm.at[idx_v.at[pl.ds(off, CH)]], rows_v, sem).wait()
            pltpu.sync_copy(rows_v, out_hbm.at[pl.ds(base + off, CH)])
            return carry

        lax.fori_loop(0, n_ch, body, 0)

    return k(table, idx)


def _dot(x, w):
    return jnp.dot(x.astype(_BF), w, preferred_element_type=jnp.float32)


def _unpack(u):
    lo = lax.bitcast_convert_type(u << 16, jnp.float32)
    hi = lax.bitcast_convert_type(u & jnp.int32(-65536), jnp.float32)
    s = jnp.sum(lo, -1, keepdims=True) + jnp.sum(hi, -1, keepdims=True)
    q = (jnp.sum(lo * lo, -1, keepdims=True)
         + jnp.sum(hi * hi, -1, keepdims=True))
    x = jnp.concatenate([lo.astype(_BF), hi.astype(_BF)], axis=-1)
    return x, s, q


def _moments(sums, ssqs, d):
    mu = sums / d
    inv = lax.rsqrt(ssqs / d - mu * mu + _EPS)
    return mu * inv, inv


def _sum2(x):
    return jnp.sum(x, -1, keepdims=True), jnp.sum(x * x, -1, keepdims=True)


def _tc_body(ni_ref, nj_ref, e_ref, nd_ref,
             w0a, w0b, w0e, u0, v0,
             w1, u1, v1,
             wl, ul, vl,
             wtr1, ut1, vt1,
             wupa, wupb, u2, v2,
             wt2a, wt2b, u3, v3,
             wue, wut, u4, v4,
             nup_ref, eup_ref):
    ni, sum_ni, ssq_ni = _unpack(ni_ref[...])
    nj, sum_nj, ssq_nj = _unpack(nj_ref[...])
    e = e_ref[...]
    C = ni.shape[1]
    DE = e.shape[1]
    BE = ni.shape[0]
    BN = nup_ref.shape[0]

    sum_e, ssq_e = _sum2(e)

    mi0, inv0 = _moments(sum_ni + sum_nj + sum_e, ssq_ni + ssq_nj + ssq_e,
                         2 * C + DE)
    x = (_dot(ni, w0a[...]) + _dot(nj, w0b[...]) + _dot(e, w0e[...]))
    x = jax.nn.relu(x * inv0 - mi0 * u0[...] + v0[...])
    s, q = _sum2(x)
    mi1, inv1 = _moments(s, q, C)
    x = jax.nn.relu(_dot(x, w1[...]) * inv1 - mi1 * u1[...] + v1[...])
    s, q = _sum2(x)
    mil, invl = _moments(s, q, C)
    m = jax.nn.relu(_dot(x, wl[...]) * invl - mil * ul[...] + vl[...])
    m_i = jnp.sum(m.reshape(BN, BE // BN, C), axis=1)

    nd = nd_ref[...]
    s, q = _sum2(nd)
    min_, invn = _moments(s, q, C)
    h = jax.nn.relu(_dot(nd, wtr1[...]) * invn - min_ * ut1[...] + vt1[...])
    sh, qh = _sum2(h)
    sm, qm = _sum2(m_i)
    mi2, inv2 = _moments(sh + sm, qh + qm, 2 * C)
    nup = jax.nn.relu(
        (_dot(h, wupa[...]) + _dot(m_i, wupb[...])) * inv2
        - mi2 * u2[...] + v2[...])
    nup_ref[...] = nup

    mi3, inv3 = _moments(sum_ni + sum_nj, ssq_ni + ssq_nj, 2 * C)
    t = jax.nn.relu(
        (_dot(ni, wt2a[...]) + _dot(nj, wt2b[...])) * inv3
        - mi3 * u3[...] + v3[...])
    st, qt = _sum2(t)
    mi4, inv4 = _moments(sum_e + st, ssq_e + qt, DE + C)
    eup = jax.nn.relu(
        (_dot(e, wue[...]) + _dot(t, wut[...])) * inv4
        - mi4 * u4[...] + v4[...])
    eup_ref[...] = eup


def _fold(s, o, w, b):
    wp = w * s[:, None]
    u = jnp.sum(wp, axis=0).reshape(1, -1)
    v = (o @ w + b).reshape(1, -1)
    return wp, u, v


def kernel(nodes, edges, i, j, params):
    N, C = nodes.shape
    E, DE = edges.shape
    H = C // 2
    P = params

    idx = jnp.concatenate([i, j]).astype(jnp.int32)
    nb = nodes.astype(_BF)
    table = lax.bitcast_convert_type(
        jnp.stack([nb[:, :H], nb[:, H:]], axis=-1), jnp.int32)
    gath = _sc_gather(table, idx)

    bf = lambda w: w.astype(_BF)
    w0p, u0, v0 = _fold(P["msg_ln0"]["s"], P["msg_ln0"]["o"],
                        P["msg_l0"]["w"], P["msg_l0"]["b"])
    w1p, u1, v1 = _fold(P["msg_ln1"]["s"], P["msg_ln1"]["o"],
                        P["msg_l1"]["w"], P["msg_l1"]["b"])
    wlp, ul, vl = _fold(P["msg_lnl"]["s"], P["msg_lnl"]["o"],
                        P["msg_ll"]["w"], P["msg_ll"]["b"])
    wt1p, ut1, vt1 = _fold(P["ln1"]["s"], P["ln1"]["o"],
                           P["tr1"]["w"], P["tr1"]["b"])
    wupp, u2, v2 = _fold(P["ln2"]["s"], P["ln2"]["o"],
                         P["up"]["w"], P["up"]["b"])
    wt2p, u3, v3 = _fold(P["ln3"]["s"], P["ln3"]["o"],
                         P["tr2"]["w"], P["tr2"]["b"])
    wuep, u4, v4 = _fold(P["ln4"]["s"], P["ln4"]["o"],
                         P["eup"]["w"], P["eup"]["b"])
    plist = [
        bf(w0p[:C]), bf(w0p[C:2 * C]), bf(w0p[2 * C:]), u0, v0,
        bf(w1p), u1, v1,
        bf(wlp), ul, vl,
        bf(wt1p), ut1, vt1,
        bf(wupp[:C]), bf(wupp[C:]), u2, v2,
        bf(wt2p[:C]), bf(wt2p[C:]), u3, v3,
        bf(wuep[:DE]), bf(wuep[DE:]), u4, v4,
    ]

    BE = 3200
    BN = BE // (E // N)
    K = 5
    SE = E // K
    Gk = SE // BE
    full = lambda p: pl.BlockSpec(p.shape, lambda b: tuple(0 for _ in p.shape))
    hbm = pl.BlockSpec(memory_space=pltpu.MemorySpace.HBM)
    n_main = 4 + len(plist)

    def chunk_body(*refs):
        _tc_body(*(refs[:n_main] + refs[n_main + 2:]))

    gaths = []
    for k in range(K):
        idx_k = jnp.concatenate(
            [i[k * SE:(k + 1) * SE], j[k * SE:(k + 1) * SE]]).astype(jnp.int32)
        gaths.append(_sc_gather(table, idx_k))

    n_up = e_up = None
    for k in range(K):
        base = k * Gk
        in_specs = [
            pl.BlockSpec((BE, H), lambda b: (b, 0)),
            pl.BlockSpec((BE, H), lambda b: (b + Gk, 0)),
            pl.BlockSpec((BE, DE), lambda b, base=base: (base + b, 0)),
            pl.BlockSpec((BN, C), lambda b, base=base: (base + b, 0)),
        ] + [full(p) for p in plist]
        out_specs = [
            pl.BlockSpec((BN, C), lambda b, base=base: (base + b, 0)),
            pl.BlockSpec((BE, C), lambda b, base=base: (base + b, 0)),
        ]
        args = [gaths[k], gaths[k], edges, nodes] + plist
        kw = {}
        if k == 0:
            body = _tc_body
        else:
            body = chunk_body
            in_specs += [hbm, hbm]
            args += [n_up, e_up]
            kw["input_output_aliases"] = {n_main: 0, n_main + 1: 1}
        n_up, e_up = pl.pallas_call(
            body,
            grid=(Gk,),
            in_specs=in_specs,
            out_specs=out_specs,
            out_shape=[
                jax.ShapeDtypeStruct((N, C), jnp.float32),
                jax.ShapeDtypeStruct((E, C), jnp.float32),
            ],
            **kw,
        )(*args)
    return (n_up, e_up)
